# baseline (speedup 1.0000x reference)
import functools

import jax
import jax.numpy as jnp
from jax import lax
from jax.experimental import pallas as pl
from jax.experimental.pallas import tpu as pltpu
from jax.experimental.pallas import tpu_sc as plsc

N = 10000
M = 2500
M_PAD = 2560
CIN = 512
C = 256
KNN = 3
RB = 200
SEG = 256
L = 16
RC = 40
NW = 32
NCHUNK = N // RC


def _hsub_body(xs_ref, w_ref, b_ref, ps_ref, o_ref, pt_ref):
    o_ref[...] = jnp.maximum(
        jnp.dot(xs_ref[...], w_ref[...], preferred_element_type=jnp.float32)
        + b_ref[...], 0.0)
    t = jnp.concatenate(
        [jnp.transpose(ps_ref[...]),
         jnp.full((3, M_PAD - M), 1e3, jnp.float32)], axis=1)
    qn = (t[0:1, :] * t[0:1, :] + t[1:2, :] * t[1:2, :]) + t[2:3, :] * t[2:3, :]
    pt_ref[...] = jnp.concatenate([t, qn], axis=0)


def _topk_body(pos_ref, psub_ref, x_ref, w_ref, b_ref, y_ref, idx_ref, wn_ref):
    y_ref[...] = jnp.maximum(
        jnp.dot(x_ref[...], w_ref[...], preferred_element_type=jnp.float32)
        + b_ref[...], 0.0)
    p = pos_ref[...]
    pn = (p[:, 0:1] * p[:, 0:1] + p[:, 1:2] * p[:, 1:2]) + p[:, 2:3] * p[:, 2:3]
    dist = (jnp.dot(p * jnp.float32(-2.0), psub_ref[0:3, :],
                    preferred_element_type=jnp.float32,
                    precision=jax.lax.Precision.HIGHEST)
            + pn) + psub_ref[3:4, :]
    iota = lax.broadcasted_iota(jnp.int32, (RB, M_PAD), 1).astype(jnp.float32)
    inf = jnp.float32(jnp.inf)
    pid = pl.program_id(0)
    idxs, ws = [], []
    for _ in range(KNN):
        m = jnp.min(dist, axis=1, keepdims=True)
        ji = jnp.min(jnp.where(dist == m, iota, jnp.float32(M_PAD)),
                     axis=1, keepdims=True)
        idxs.append(ji)
        ws.append(1.0 / jnp.maximum(m, jnp.float32(1e-16)))
        dist = jnp.where(iota == ji, inf, dist)
    den = (ws[0] + ws[1]) + ws[2]
    for k in range(KNN):
        seg = pl.ds((pid * KNN + k) * SEG, RB)
        idx_ref[seg] = jnp.transpose(idxs[k]).reshape(RB).astype(jnp.int32)
        wn_ref[seg] = jnp.transpose(ws[k] / den).reshape(RB)


_hsub_call = pl.pallas_call(
    _hsub_body,
    out_shape=[
        jax.ShapeDtypeStruct((M, C), jnp.float32),
        jax.ShapeDtypeStruct((4, M_PAD), jnp.float32),
    ],
)

@functools.cache
def _topk_call(nh, off):
    return pl.pallas_call(
        _topk_body,
        grid=(nh // RB,),
        in_specs=[
            pl.BlockSpec((RB, 3), lambda i: (i + off, 0)),
            pl.BlockSpec((4, M_PAD), lambda i: (0, 0)),
            pl.BlockSpec((RB, C), lambda i: (i + off, 0)),
            pl.BlockSpec((C, C), lambda i: (0, 0)),
            pl.BlockSpec((1, C), lambda i: (0, 0)),
        ],
        out_specs=[
            pl.BlockSpec((RB, C), lambda i: (i, 0)),
            pl.BlockSpec((nh // RB * KNN * SEG,), lambda i: (0,)),
            pl.BlockSpec((nh // RB * KNN * SEG,), lambda i: (0,)),
        ],
        out_shape=[
            jax.ShapeDtypeStruct((nh, C), jnp.float32),
            jax.ShapeDtypeStruct((nh // RB * KNN * SEG,), jnp.int32),
            jax.ShapeDtypeStruct((nh // RB * KNN * SEG,), jnp.float32),
        ],
    )


def _make_sc_body(nchunk, nh):
  nloop = (nchunk + NW - 1) // NW

  def _sc_body(h_hbm, idxf_hbm, w_hbm, y_hbm, out_hbm,
               idx_all, g_v, w_v, y_v, out_v,
               sem_i, sem_g0, sem_g1, sem_w0, sem_w1, sem_y0, sem_y1, sem_o):
    sems_g = [sem_g0, sem_g1]
    sems_w = [sem_w0, sem_w1]
    sems_y = [sem_y0, sem_y1]
    wid = lax.axis_index("s") * 2 + lax.axis_index("c")

    def seg_offs(ch):
        tb = lax.div(ch, RB // RC)
        within = lax.rem(ch, RB // RC) * RC
        return [(tb * KNN + k) * SEG + within for k in range(KNN)]

    for j in range(nloop):
        ch = wid + j * NW

        @pl.when(ch < nchunk)
        def _(j=j, ch=ch):
            for k, so in enumerate(seg_offs(ch)):
                pltpu.async_copy(idxf_hbm.at[pl.ds(so, RC)],
                                 idx_all.at[j, k], sem_i)

    for j in range(nloop):
        ch = wid + j * NW

        @pl.when(ch < nchunk)
        def _(j=j, ch=ch):
            for k in range(KNN):
                pltpu.make_async_copy(idxf_hbm.at[pl.ds(0, RC)],
                                      idx_all.at[j, k], sem_i).wait()

    def fire(j, b):
        if j >= nloop:
            return
        ch = wid + j * NW

        @pl.when(ch < nchunk)
        def _():
            base = ch * RC
            for k, so in enumerate(seg_offs(ch)):
                pltpu.async_copy(h_hbm.at[idx_all.at[j, k]],
                                 g_v.at[b, k], sems_g[b])
                pltpu.async_copy(w_hbm.at[pl.ds(so, RC)],
                                 w_v.at[b, k, pl.ds(0, RC)], sems_w[b])
            pltpu.async_copy(y_hbm.at[pl.ds(base, RC)], y_v.at[b], sems_y[b])

    def consume(j, b):
        ch = wid + j * NW

        @pl.when(ch < nchunk)
        def _():
            base = ch * RC
            for k in range(KNN):
                pltpu.make_async_copy(h_hbm.at[idx_all.at[j, k]],
                                      g_v.at[b, k], sems_g[b]).wait()
                pltpu.make_async_copy(w_hbm.at[pl.ds(0, RC)],
                                      w_v.at[b, k, pl.ds(0, RC)],
                                      sems_w[b]).wait()
            pltpu.make_async_copy(y_hbm.at[pl.ds(0, RC)],
                                  y_v.at[b], sems_y[b]).wait()

            def grp_body(g, c2):
                g8 = g * 8
                wg = [w_v[b, k, pl.ds(g8, L)] for k in range(KNN)]
                for u in range(8):
                    r = g8 + u
                    wu = [wg[k][u] for k in range(KNN)]
                    for cc in range(C // L):
                        sl = pl.ds(cc * L, L)
                        acc = y_v[b, r, sl]
                        for k in range(KNN):
                            acc = acc + wu[k] * g_v[b, k, r, sl]
                        out_v[b, r, sl] = acc
                return c2

            lax.fori_loop(0, RC // 8, grp_body, 0)
            @pl.when(j >= 2)
            def _():
                pltpu.make_async_copy(
                    out_v.at[b], out_hbm.at[pl.ds(base, RC)], sem_o).wait()

            pltpu.async_copy(out_v.at[b], out_hbm.at[pl.ds(base, RC)], sem_o)

    fire(0, 0)
    for j in range(nloop):
        b = j % 2
        fire(j + 1, 1 - b)
        consume(j, b)

    na = lax.div(nchunk - wid + NW - 1, NW)

    @pl.when(na >= 1)
    def _():
        pltpu.make_async_copy(
            out_v.at[0], out_hbm.at[pl.ds(0, RC)], sem_o).wait()

    @pl.when(na >= 2)
    def _():
        pltpu.make_async_copy(
            out_v.at[0], out_hbm.at[pl.ds(0, RC)], sem_o).wait()

  return _sc_body


@functools.cache
def _sc_call(nh):
    return pl.kernel(
        _make_sc_body(nh // RC, nh),
        out_type=jax.ShapeDtypeStruct((nh, C), jnp.float32),
        mesh=plsc.VectorSubcoreMesh(core_axis_name="c", subcore_axis_name="s"),
        scratch_types=[
            pltpu.VMEM(((nh // RC + NW - 1) // NW, KNN, RC), jnp.int32),
            pltpu.VMEM((2, KNN, RC, C), jnp.float32),
            pltpu.VMEM((2, KNN, RC + L), jnp.float32),
            pltpu.VMEM((2, RC, C), jnp.float32),
            pltpu.VMEM((2, RC, C), jnp.float32),
        ] + [pltpu.SemaphoreType.DMA] * 8,
    )


NH = N // 2


def kernel(x, x_sub, pos, pos_sub, W_sub, b_sub, W, b):
    h_sub, psubT = _hsub_call(x_sub, W_sub, b_sub.reshape(1, C), pos_sub)
    b1 = b.reshape(1, C)
    outs = []
    for p in range(N // NH):
        y, idx_flat, w_flat = _topk_call(NH, p * (NH // RB))(
            pos, psubT, x, W, b1)
        outs.append(_sc_call(NH)(h_sub, idx_flat, w_flat, y))
    buf = jnp.pad(outs[0], ((0, N - NH), (0, 0)))
    return lax.dynamic_update_slice(buf, outs[1], (NH, 0))


# speedup vs baseline: 1.3860x; 1.3860x over previous
import functools

import jax
import jax.numpy as jnp
from jax import lax
from jax.experimental import pallas as pl
from jax.experimental.pallas import tpu as pltpu
from jax.experimental.pallas import tpu_sc as plsc

N = 10000
M = 2500
M_PAD = 2560
CIN = 512
C = 256
KNN = 3
RB = 200
SEG = 256
L = 16
RC = 40
NW = 32
NCHUNK = N // RC


def _hsub_body(xs_ref, w_ref, b_ref, ps_ref, o_ref, pt_ref):
    o_ref[...] = jnp.maximum(
        jnp.dot(xs_ref[...], w_ref[...], preferred_element_type=jnp.float32)
        + b_ref[...], 0.0)
    t = jnp.concatenate(
        [jnp.transpose(ps_ref[...]),
         jnp.full((3, M_PAD - M), 1e3, jnp.float32)], axis=1)
    qn = (t[0:1, :] * t[0:1, :] + t[1:2, :] * t[1:2, :]) + t[2:3, :] * t[2:3, :]
    pt_ref[...] = jnp.concatenate([t, qn], axis=0)


def _topk_body(pos_ref, psub_ref, x_ref, w_ref, b_ref, y_ref, idx_ref, wn_ref):
    y_ref[...] = jnp.maximum(
        jnp.dot(x_ref[...], w_ref[...], preferred_element_type=jnp.float32)
        + b_ref[...], 0.0)
    p = pos_ref[...]
    d0 = p[:, 0:1] - psub_ref[0:1, :]
    d1 = p[:, 1:2] - psub_ref[1:2, :]
    d2 = p[:, 2:3] - psub_ref[2:3, :]
    dist = (d0 * d0 + d1 * d1) + d2 * d2
    iota = lax.broadcasted_iota(jnp.int32, (RB, M_PAD), 1).astype(jnp.float32)
    inf = jnp.float32(jnp.inf)
    pid = pl.program_id(0)
    idxs, ws = [], []
    for _ in range(KNN):
        m = jnp.min(dist, axis=1, keepdims=True)
        ji = jnp.min(jnp.where(dist == m, iota, jnp.float32(M_PAD)),
                     axis=1, keepdims=True)
        idxs.append(ji)
        ws.append(1.0 / jnp.maximum(m, jnp.float32(1e-16)))
        dist = jnp.where(iota == ji, inf, dist)
    den = (ws[0] + ws[1]) + ws[2]
    for k in range(KNN):
        seg = pl.ds((pid * KNN + k) * SEG, RB)
        idx_ref[seg] = jnp.transpose(idxs[k]).reshape(RB).astype(jnp.int32)
        wn_ref[seg] = jnp.transpose(ws[k] / den).reshape(RB)


_hsub_call = pl.pallas_call(
    _hsub_body,
    out_shape=[
        jax.ShapeDtypeStruct((M, C), jnp.float32),
        jax.ShapeDtypeStruct((4, M_PAD), jnp.float32),
    ],
)

@functools.cache
def _topk_call(nh, off):
    return pl.pallas_call(
        _topk_body,
        grid=(nh // RB,),
        in_specs=[
            pl.BlockSpec((RB, 3), lambda i: (i + off, 0)),
            pl.BlockSpec((4, M_PAD), lambda i: (0, 0)),
            pl.BlockSpec((RB, C), lambda i: (i + off, 0)),
            pl.BlockSpec((C, C), lambda i: (0, 0)),
            pl.BlockSpec((1, C), lambda i: (0, 0)),
        ],
        out_specs=[
            pl.BlockSpec((RB, C), lambda i: (i, 0)),
            pl.BlockSpec((nh // RB * KNN * SEG,), lambda i: (0,)),
            pl.BlockSpec((nh // RB * KNN * SEG,), lambda i: (0,)),
        ],
        out_shape=[
            jax.ShapeDtypeStruct((nh, C), jnp.float32),
            jax.ShapeDtypeStruct((nh // RB * KNN * SEG,), jnp.int32),
            jax.ShapeDtypeStruct((nh // RB * KNN * SEG,), jnp.float32),
        ],
    )


def _make_sc_body(nchunk, nh):
  nloop = (nchunk + NW - 1) // NW

  def _sc_body(h_hbm, idxf_hbm, w_hbm, y_hbm, out_hbm,
               idx_all, g_v, w_v, y_v, out_v,
               sem_i, sem_g0, sem_g1, sem_w0, sem_w1, sem_y0, sem_y1, sem_o):
    sems_g = [sem_g0, sem_g1]
    sems_w = [sem_w0, sem_w1]
    sems_y = [sem_y0, sem_y1]
    wid = lax.axis_index("s") * 2 + lax.axis_index("c")

    def seg_offs(ch):
        tb = lax.div(ch, RB // RC)
        within = lax.rem(ch, RB // RC) * RC
        return [(tb * KNN + k) * SEG + within for k in range(KNN)]

    for j in range(nloop):
        ch = wid + j * NW

        @pl.when(ch < nchunk)
        def _(j=j, ch=ch):
            for k, so in enumerate(seg_offs(ch)):
                pltpu.async_copy(idxf_hbm.at[pl.ds(so, RC)],
                                 idx_all.at[j, k], sem_i)

    for j in range(nloop):
        ch = wid + j * NW

        @pl.when(ch < nchunk)
        def _(j=j, ch=ch):
            for k in range(KNN):
                pltpu.make_async_copy(idxf_hbm.at[pl.ds(0, RC)],
                                      idx_all.at[j, k], sem_i).wait()

    def fire(j, b):
        if j >= nloop:
            return
        ch = wid + j * NW

        @pl.when(ch < nchunk)
        def _():
            base = ch * RC
            for k, so in enumerate(seg_offs(ch)):
                pltpu.async_copy(h_hbm.at[idx_all.at[j, k]],
                                 g_v.at[b, k], sems_g[b])
                pltpu.async_copy(w_hbm.at[pl.ds(so, RC)],
                                 w_v.at[b, k, pl.ds(0, RC)], sems_w[b])
            pltpu.async_copy(y_hbm.at[pl.ds(base, RC)], y_v.at[b], sems_y[b])

    def consume(j, b):
        ch = wid + j * NW

        @pl.when(ch < nchunk)
        def _():
            base = ch * RC
            for k in range(KNN):
                pltpu.make_async_copy(h_hbm.at[idx_all.at[j, k]],
                                      g_v.at[b, k], sems_g[b]).wait()
                pltpu.make_async_copy(w_hbm.at[pl.ds(0, RC)],
                                      w_v.at[b, k, pl.ds(0, RC)],
                                      sems_w[b]).wait()
            pltpu.make_async_copy(y_hbm.at[pl.ds(0, RC)],
                                  y_v.at[b], sems_y[b]).wait()

            def grp_body(g, c2):
                g8 = g * 8
                wg = [w_v[b, k, pl.ds(g8, L)] for k in range(KNN)]
                for u in range(8):
                    r = g8 + u
                    wu = [wg[k][u] for k in range(KNN)]
                    for cc in range(C // L):
                        sl = pl.ds(cc * L, L)
                        acc = y_v[b, r, sl]
                        for k in range(KNN):
                            acc = acc + wu[k] * g_v[b, k, r, sl]
                        out_v[b, r, sl] = acc
                return c2

            lax.fori_loop(0, RC // 8, grp_body, 0)
            @pl.when(j >= 2)
            def _():
                pltpu.make_async_copy(
                    out_v.at[b], out_hbm.at[pl.ds(base, RC)], sem_o).wait()

            pltpu.async_copy(out_v.at[b], out_hbm.at[pl.ds(base, RC)], sem_o)

    fire(0, 0)
    for j in range(nloop):
        b = j % 2
        fire(j + 1, 1 - b)
        consume(j, b)

    na = lax.div(nchunk - wid + NW - 1, NW)

    @pl.when(na >= 1)
    def _():
        pltpu.make_async_copy(
            out_v.at[0], out_hbm.at[pl.ds(0, RC)], sem_o).wait()

    @pl.when(na >= 2)
    def _():
        pltpu.make_async_copy(
            out_v.at[0], out_hbm.at[pl.ds(0, RC)], sem_o).wait()

  return _sc_body


@functools.cache
def _sc_call(nh):
    return pl.kernel(
        _make_sc_body(nh // RC, nh),
        out_type=jax.ShapeDtypeStruct((nh, C), jnp.float32),
        mesh=plsc.VectorSubcoreMesh(core_axis_name="c", subcore_axis_name="s"),
        scratch_types=[
            pltpu.VMEM(((nh // RC + NW - 1) // NW, KNN, RC), jnp.int32),
            pltpu.VMEM((2, KNN, RC, C), jnp.float32),
            pltpu.VMEM((2, KNN, RC + L), jnp.float32),
            pltpu.VMEM((2, RC, C), jnp.float32),
            pltpu.VMEM((2, RC, C), jnp.float32),
        ] + [pltpu.SemaphoreType.DMA] * 8,
    )


NH = N // 5


def kernel(x, x_sub, pos, pos_sub, W_sub, b_sub, W, b):
    h_sub, psubT = _hsub_call(x_sub, W_sub, b_sub.reshape(1, C), pos_sub)
    b1 = b.reshape(1, C)
    outs = []
    for p in range(N // NH):
        y, idx_flat, w_flat = _topk_call(NH, p * (NH // RB))(
            pos, psubT, x, W, b1)
        outs.append(_sc_call(NH)(h_sub, idx_flat, w_flat, y))
    buf = jnp.pad(outs[0], ((0, N - NH), (0, 0)))
    for p in range(1, N // NH):
        buf = lax.dynamic_update_slice(buf, outs[p], (p * NH, 0))
    return buf


# speedup vs baseline: 1.4376x; 1.0372x over previous
import functools

import jax
import jax.numpy as jnp
from jax import lax
from jax.experimental import pallas as pl
from jax.experimental.pallas import tpu as pltpu
from jax.experimental.pallas import tpu_sc as plsc

N = 10000
M = 2500
M_PAD = 2560
CIN = 512
C = 256
KNN = 3
RB = 200
SEG = 256
L = 16
RC = 40
NW = 32
NCHUNK = N // RC


def _hsub_body(xs_ref, w_ref, b_ref, pst_ref, pt_ref2, o_ref, pt_ref, pos_ref):
    o_ref[...] = jnp.maximum(
        jnp.dot(xs_ref[...], w_ref[...], preferred_element_type=jnp.float32)
        + b_ref[...], 0.0)
    pt_ref[...] = jnp.concatenate(
        [pst_ref[...], jnp.full((3, M_PAD - M), 1e3, jnp.float32)], axis=1)
    pos_ref[...] = jnp.transpose(pt_ref2[...])


def _topk_body(pos_ref, psub_ref, x_ref, w_ref, b_ref, y_ref, idx_ref, wn_ref):
    y_ref[...] = jnp.maximum(
        jnp.dot(x_ref[...], w_ref[...], preferred_element_type=jnp.float32)
        + b_ref[...], 0.0)
    p = pos_ref[...]
    d0 = p[:, 0:1] - psub_ref[0:1, :]
    d1 = p[:, 1:2] - psub_ref[1:2, :]
    d2 = p[:, 2:3] - psub_ref[2:3, :]
    dist = (d0 * d0 + d1 * d1) + d2 * d2
    iota = lax.broadcasted_iota(jnp.int32, (RB, M_PAD), 1).astype(jnp.float32)
    inf = jnp.float32(jnp.inf)
    pid = pl.program_id(0)
    idxs, ws = [], []
    for _ in range(KNN):
        m = jnp.min(dist, axis=1, keepdims=True)
        ji = jnp.min(jnp.where(dist == m, iota, jnp.float32(M_PAD)),
                     axis=1, keepdims=True)
        idxs.append(ji)
        ws.append(1.0 / jnp.maximum(m, jnp.float32(1e-16)))
        dist = jnp.where(iota == ji, inf, dist)
    den = (ws[0] + ws[1]) + ws[2]
    for k in range(KNN):
        seg = pl.ds((pid * KNN + k) * SEG, RB)
        idx_ref[seg] = jnp.transpose(idxs[k]).reshape(RB).astype(jnp.int32)
        wn_ref[seg] = jnp.transpose(ws[k] / den).reshape(RB)


_hsub_call = pl.pallas_call(
    _hsub_body,
    out_shape=[
        jax.ShapeDtypeStruct((M, C), jnp.float32),
        jax.ShapeDtypeStruct((3, M_PAD), jnp.float32),
        jax.ShapeDtypeStruct((N, 3), jnp.float32),
    ],
)

@functools.cache
def _topk_call(nh, off):
    return pl.pallas_call(
        _topk_body,
        grid=(nh // RB,),
        in_specs=[
            pl.BlockSpec((RB, 3), lambda i: (i + off, 0)),
            pl.BlockSpec((3, M_PAD), lambda i: (0, 0)),
            pl.BlockSpec((RB, C), lambda i: (i + off, 0)),
            pl.BlockSpec((C, C), lambda i: (0, 0)),
            pl.BlockSpec((1, C), lambda i: (0, 0)),
        ],
        out_specs=[
            pl.BlockSpec((RB, C), lambda i: (i, 0)),
            pl.BlockSpec((nh // RB * KNN * SEG,), lambda i: (0,)),
            pl.BlockSpec((nh // RB * KNN * SEG,), lambda i: (0,)),
        ],
        out_shape=[
            jax.ShapeDtypeStruct((nh, C), jnp.float32),
            jax.ShapeDtypeStruct((nh // RB * KNN * SEG,), jnp.int32),
            jax.ShapeDtypeStruct((nh // RB * KNN * SEG,), jnp.float32),
        ],
    )


def _make_sc_body(nchunk, nh):
  nloop = (nchunk + NW - 1) // NW

  def _sc_body(h_hbm, idxf_hbm, w_hbm, y_hbm, out_hbm,
               idx_all, g_v, w_v, y_v, out_v,
               sem_i, sem_g0, sem_g1, sem_w0, sem_w1, sem_y0, sem_y1, sem_o):
    sems_g = [sem_g0, sem_g1]
    sems_w = [sem_w0, sem_w1]
    sems_y = [sem_y0, sem_y1]
    wid = lax.axis_index("s") * 2 + lax.axis_index("c")

    def seg_offs(ch):
        tb = lax.div(ch, RB // RC)
        within = lax.rem(ch, RB // RC) * RC
        return [(tb * KNN + k) * SEG + within for k in range(KNN)]

    for j in range(nloop):
        ch = wid + j * NW

        @pl.when(ch < nchunk)
        def _(j=j, ch=ch):
            for k, so in enumerate(seg_offs(ch)):
                pltpu.async_copy(idxf_hbm.at[pl.ds(so, RC)],
                                 idx_all.at[j, k], sem_i)

    for j in range(nloop):
        ch = wid + j * NW

        @pl.when(ch < nchunk)
        def _(j=j, ch=ch):
            for k in range(KNN):
                pltpu.make_async_copy(idxf_hbm.at[pl.ds(0, RC)],
                                      idx_all.at[j, k], sem_i).wait()

    def fire(j, b):
        if j >= nloop:
            return
        ch = wid + j * NW

        @pl.when(ch < nchunk)
        def _():
            base = ch * RC
            for k, so in enumerate(seg_offs(ch)):
                pltpu.async_copy(h_hbm.at[idx_all.at[j, k]],
                                 g_v.at[b, k], sems_g[b])
                pltpu.async_copy(w_hbm.at[pl.ds(so, RC)],
                                 w_v.at[b, k, pl.ds(0, RC)], sems_w[b])
            pltpu.async_copy(y_hbm.at[pl.ds(base, RC)], y_v.at[b], sems_y[b])

    def consume(j, b):
        ch = wid + j * NW

        @pl.when(ch < nchunk)
        def _():
            base = ch * RC
            for k in range(KNN):
                pltpu.make_async_copy(h_hbm.at[idx_all.at[j, k]],
                                      g_v.at[b, k], sems_g[b]).wait()
                pltpu.make_async_copy(w_hbm.at[pl.ds(0, RC)],
                                      w_v.at[b, k, pl.ds(0, RC)],
                                      sems_w[b]).wait()
            pltpu.make_async_copy(y_hbm.at[pl.ds(0, RC)],
                                  y_v.at[b], sems_y[b]).wait()

            def grp_body(g, c2):
                g8 = g * 8
                wg = [w_v[b, k, pl.ds(g8, L)] for k in range(KNN)]
                for u in range(8):
                    r = g8 + u
                    wu = [wg[k][u] for k in range(KNN)]
                    for cc in range(C // L):
                        sl = pl.ds(cc * L, L)
                        acc = y_v[b, r, sl]
                        for k in range(KNN):
                            acc = acc + wu[k] * g_v[b, k, r, sl]
                        out_v[b, r, sl] = acc
                return c2

            lax.fori_loop(0, RC // 8, grp_body, 0)
            @pl.when(j >= 2)
            def _():
                pltpu.make_async_copy(
                    out_v.at[b], out_hbm.at[pl.ds(base, RC)], sem_o).wait()

            pltpu.async_copy(out_v.at[b], out_hbm.at[pl.ds(base, RC)], sem_o)

    fire(0, 0)
    for j in range(nloop):
        b = j % 2
        fire(j + 1, 1 - b)
        consume(j, b)

    na = lax.div(nchunk - wid + NW - 1, NW)

    @pl.when(na >= 1)
    def _():
        pltpu.make_async_copy(
            out_v.at[0], out_hbm.at[pl.ds(0, RC)], sem_o).wait()

    @pl.when(na >= 2)
    def _():
        pltpu.make_async_copy(
            out_v.at[0], out_hbm.at[pl.ds(0, RC)], sem_o).wait()

  return _sc_body


@functools.cache
def _sc_call(nh, out_rows):
    return pl.kernel(
        _make_sc_body(nh // RC, nh),
        out_type=jax.ShapeDtypeStruct((out_rows, C), jnp.float32),
        mesh=plsc.VectorSubcoreMesh(core_axis_name="c", subcore_axis_name="s"),
        scratch_types=[
            pltpu.VMEM(((nh // RC + NW - 1) // NW, KNN, RC), jnp.int32),
            pltpu.VMEM((2, KNN, RC, C), jnp.float32),
            pltpu.VMEM((2, KNN, RC + L), jnp.float32),
            pltpu.VMEM((2, RC, C), jnp.float32),
            pltpu.VMEM((2, RC, C), jnp.float32),
        ] + [pltpu.SemaphoreType.DMA] * 8,
    )


NH = N // 5


def kernel(x, x_sub, pos, pos_sub, W_sub, b_sub, W, b):
    h_sub, psubT, pos_rm = _hsub_call(
        x_sub, W_sub, b_sub.reshape(1, C), pos_sub.T, pos.T)
    b1 = b.reshape(1, C)
    outs = []
    for p in range(N // NH):
        y, idx_flat, w_flat = _topk_call(NH, p * (NH // RB))(
            pos_rm, psubT, x, W, b1)
        outs.append(_sc_call(NH, N if p == 0 else NH)(
            h_sub, idx_flat, w_flat, y))
    buf = outs[0]
    for p in range(1, N // NH):
        buf = lax.dynamic_update_slice(buf, outs[p], (p * NH, 0))
    return buf
